# baseline (speedup 1.0000x reference)
import jax
import jax.numpy as jnp
from jax.experimental import pallas as pl
from jax.experimental.pallas import tpu as pltpu


def _seg_sum_kernel(x0_ref, x1_ref, x2_ref, x3_ref, m_ref, sums_ref, area_ref):
    k = pl.program_id(1)

    @pl.when(k == 0)
    def _():
        sums_ref[...] = jnp.zeros_like(sums_ref)
        area_ref[...] = jnp.zeros_like(area_ref)

    mm = m_ref[0]
    CQ = x0_ref.shape[1]
    for i, xr in enumerate((x0_ref, x1_ref, x2_ref, x3_ref)):
        xm = xr[0].astype(jnp.bfloat16)
        part = jax.lax.dot_general(
            xm, mm, (((1,), (1,)), ((), ())),
            preferred_element_type=jnp.float32)
        sums_ref[0, i * CQ:(i + 1) * CQ, :] += part
    area_ref[0] += jnp.broadcast_to(
        jnp.sum(mm.astype(jnp.float32), axis=1)[:, None], area_ref.shape[1:])


def _linear_kernel(s_ref, a_ref, w_ref, b_ref, o_ref):
    area = a_ref[0, 0, :]
    feat = s_ref[0]
    feat = jnp.maximum(feat / jnp.maximum(area, 1.0)[:, None], 0.0)
    o = jax.lax.dot_general(
        feat, w_ref[0], (((1,), (1,)), ((), ())),
        preferred_element_type=jnp.float32)
    o = o + b_ref[0]
    o_ref[0] = jnp.where(area[:, None] > 0.0, o, 0.0)


@jax.jit
def kernel(x, segmap_attentions, W, b):
    B, C, H, Wsp = x.shape
    J, MH, MW = segmap_attentions.shape[1:]
    fh, fw = H // MH, Wsp // MW
    HW = H * Wsp

    m = (segmap_attentions != 0).astype(jnp.bfloat16)
    m = jnp.repeat(jnp.repeat(m, fh, axis=2), fw, axis=3).reshape(B, J, HW)
    xf = x.reshape(B, C, HW)

    TK = 24576 if HW % 24576 == 0 else HW
    nK = HW // TK

    sums_cj, area = pl.pallas_call(
        _seg_sum_kernel,
        grid=(B, nK),
        in_specs=[
            pl.BlockSpec((1, C // 4, TK), lambda bb, k: (bb, 0, k)),
            pl.BlockSpec((1, C // 4, TK), lambda bb, k: (bb, 1, k)),
            pl.BlockSpec((1, C // 4, TK), lambda bb, k: (bb, 2, k)),
            pl.BlockSpec((1, C // 4, TK), lambda bb, k: (bb, 3, k)),
            pl.BlockSpec((1, J, TK), lambda bb, k: (bb, 0, k)),
        ],
        out_specs=(
            pl.BlockSpec((1, C, J), lambda bb, k: (bb, 0, 0)),
            pl.BlockSpec((1, J, 128), lambda bb, k: (bb, 0, 0)),
        ),
        out_shape=(
            jax.ShapeDtypeStruct((B, C, J), jnp.float32),
            jax.ShapeDtypeStruct((B, J, 128), jnp.float32),
        ),
        compiler_params=pltpu.CompilerParams(
            dimension_semantics=("parallel", "arbitrary")),
    )(xf, xf, xf, xf, m)

    sums_t = sums_cj.transpose(2, 0, 1)
    area_t = area[:, :, 0].transpose(1, 0).reshape(J, 1, B)
    b2 = b.reshape(J, 1, C)

    out_t = pl.pallas_call(
        _linear_kernel,
        grid=(J,),
        in_specs=[
            pl.BlockSpec((1, B, C), lambda j: (j, 0, 0)),
            pl.BlockSpec((1, 1, B), lambda j: (j, 0, 0)),
            pl.BlockSpec((1, C, C), lambda j: (j, 0, 0)),
            pl.BlockSpec((1, 1, C), lambda j: (j, 0, 0)),
        ],
        out_specs=pl.BlockSpec((1, B, C), lambda j: (j, 0, 0)),
        out_shape=jax.ShapeDtypeStruct((J, B, C), jnp.float32),
    )(sums_t, area_t, W, b2)

    return out_t.transpose(1, 0, 2)


# speedup vs baseline: 1.0026x; 1.0026x over previous
import jax
import jax.numpy as jnp
from jax.experimental import pallas as pl
from jax.experimental.pallas import tpu as pltpu


def _seg_sum_kernel(x_ref, m_ref, sums_ref, area_ref):
    k = pl.program_id(1)

    @pl.when(k == 0)
    def _():
        sums_ref[...] = jnp.zeros_like(sums_ref)
        area_ref[...] = jnp.zeros_like(area_ref)

    mm = m_ref[0]
    xm = x_ref[0].astype(jnp.bfloat16)
    part = jax.lax.dot_general(
        mm, xm, (((1,), (1,)), ((), ())),
        preferred_element_type=jnp.float32)
    sums_ref[0] += part
    area_ref[0] += jnp.broadcast_to(
        jnp.sum(mm.astype(jnp.float32), axis=1)[:, None], area_ref.shape[1:])


def _linear_kernel(s_ref, a_ref, w_ref, b_ref, o_ref):
    area = a_ref[0, 0, :]
    feat = s_ref[0]
    feat = jnp.maximum(feat / jnp.maximum(area, 1.0)[:, None], 0.0)
    o = jax.lax.dot_general(
        feat, w_ref[0], (((1,), (1,)), ((), ())),
        preferred_element_type=jnp.float32)
    o = o + b_ref[0]
    o_ref[0] = jnp.where(area[:, None] > 0.0, o, 0.0)


@jax.jit
def kernel(x, segmap_attentions, W, b):
    B, C, H, Wsp = x.shape
    J, MH, MW = segmap_attentions.shape[1:]
    fh, fw = H // MH, Wsp // MW
    HW = H * Wsp

    m = (segmap_attentions != 0).astype(jnp.bfloat16)
    m = jnp.repeat(jnp.repeat(m, fh, axis=2), fw, axis=3).reshape(B, J, HW)
    xf = x.reshape(B, C, HW)

    TK = 24576 if HW % 24576 == 0 else HW
    nK = HW // TK

    sums_cj, area = pl.pallas_call(
        _seg_sum_kernel,
        grid=(B, nK),
        in_specs=[
            pl.BlockSpec((1, C, TK), lambda bb, k: (bb, 0, k)),
            pl.BlockSpec((1, J, TK), lambda bb, k: (bb, 0, k)),
        ],
        out_specs=(
            pl.BlockSpec((1, J, C), lambda bb, k: (bb, 0, 0)),
            pl.BlockSpec((1, J, 128), lambda bb, k: (bb, 0, 0)),
        ),
        out_shape=(
            jax.ShapeDtypeStruct((B, J, C), jnp.float32),
            jax.ShapeDtypeStruct((B, J, 128), jnp.float32),
        ),
        compiler_params=pltpu.CompilerParams(
            dimension_semantics=("parallel", "arbitrary")),
    )(xf, m)

    sums_t = sums_cj.transpose(1, 0, 2)
    area_t = area[:, :, 0].transpose(1, 0).reshape(J, 1, B)
    b2 = b.reshape(J, 1, C)

    out_t = pl.pallas_call(
        _linear_kernel,
        grid=(J,),
        in_specs=[
            pl.BlockSpec((1, B, C), lambda j: (j, 0, 0)),
            pl.BlockSpec((1, 1, B), lambda j: (j, 0, 0)),
            pl.BlockSpec((1, C, C), lambda j: (j, 0, 0)),
            pl.BlockSpec((1, 1, C), lambda j: (j, 0, 0)),
        ],
        out_specs=pl.BlockSpec((1, B, C), lambda j: (j, 0, 0)),
        out_shape=jax.ShapeDtypeStruct((J, B, C), jnp.float32),
    )(sums_t, area_t, W, b2)

    return out_t.transpose(1, 0, 2)
